# initial kernel scaffold (unmeasured)
import jax
import jax.numpy as jnp
from jax import lax
from jax.experimental import pallas as pl
from jax.experimental.pallas import tpu as pltpu


def kernel(
    x,
):
    def body(*refs):
        pass

    out_shape = jax.ShapeDtypeStruct(..., jnp.float32)
    return pl.pallas_call(body, out_shape=out_shape)(...)


# baseline (device time: 55251 ns/iter reference)
import jax
import jax.numpy as jnp
from jax import lax
from jax.experimental import pallas as pl
from jax.experimental.pallas import tpu as pltpu

X_SIZE = 2


def kernel(x):
    m, n = x.shape
    n_out = n // X_SIZE
    m_out = m * X_SIZE

    def body(x_ref, out_ref, send_sem, recv_sem):
        my_x = lax.axis_index("x")
        my_y = lax.axis_index("y")
        other_x = 1 - my_x

        barrier_sem = pltpu.get_barrier_semaphore()
        pl.semaphore_signal(
            barrier_sem, inc=1,
            device_id=(other_x, my_y), device_id_type=pl.DeviceIdType.MESH,
        )
        pl.semaphore_wait(barrier_sem, 1)

        rdma = pltpu.make_async_remote_copy(
            src_ref=x_ref.at[:, pl.ds(other_x * n_out, n_out)],
            dst_ref=out_ref.at[pl.ds(my_x * m, m), :],
            send_sem=send_sem,
            recv_sem=recv_sem,
            device_id=(other_x, my_y),
            device_id_type=pl.DeviceIdType.MESH,
        )
        rdma.start()

        out_ref[pl.ds(my_x * m, m), :] = x_ref[:, pl.ds(my_x * n_out, n_out)]

        rdma.wait()

    return pl.pallas_call(
        body,
        out_shape=jax.ShapeDtypeStruct((m_out, n_out), x.dtype),
        in_specs=[pl.BlockSpec(memory_space=pltpu.VMEM)],
        out_specs=pl.BlockSpec(memory_space=pltpu.VMEM),
        scratch_shapes=[
            pltpu.SemaphoreType.DMA,
            pltpu.SemaphoreType.DMA,
        ],
        compiler_params=pltpu.CompilerParams(collective_id=0),
    )(x)


# device time: 38090 ns/iter; 1.4505x vs baseline; 1.4505x over previous
import jax
import jax.numpy as jnp
from jax import lax
from jax.experimental import pallas as pl
from jax.experimental.pallas import tpu as pltpu

X_SIZE = 2
N_CHUNKS = 8


def kernel(x):
    m, n = x.shape
    n_out = n // X_SIZE
    m_out = m * X_SIZE
    half = m // 2
    ck = half // N_CHUNKS

    def body(x_ref, out_ref, p1_send, p1_recv, p2_send, p2_recv):
        my_x = lax.axis_index("x")
        my_y = lax.axis_index("y")
        other_x = 1 - my_x
        other_y = 1 - my_y

        barrier_sem = pltpu.get_barrier_semaphore()
        for dev in [(other_x, my_y), (my_x, other_y)]:
            pl.semaphore_signal(
                barrier_sem, inc=1,
                device_id=dev, device_id_type=pl.DeviceIdType.MESH,
            )
        pl.semaphore_wait(barrier_sem, 2)

        p1 = []
        for i in range(N_CHUNKS):
            rdma = pltpu.make_async_remote_copy(
                src_ref=x_ref.at[
                    pl.ds(my_y * half + i * ck, ck),
                    pl.ds(other_x * n_out, n_out),
                ],
                dst_ref=out_ref.at[pl.ds(my_x * m + my_y * half + i * ck, ck), :],
                send_sem=p1_send.at[i],
                recv_sem=p1_recv.at[i],
                device_id=(other_x, my_y),
                device_id_type=pl.DeviceIdType.MESH,
            )
            rdma.start()
            p1.append(rdma)

        out_ref[pl.ds(my_x * m, m), :] = x_ref[:, pl.ds(my_x * n_out, n_out)]

        p2 = []
        for i in range(N_CHUNKS):
            p1[i].wait_recv()
            rdma = pltpu.make_async_remote_copy(
                src_ref=out_ref.at[
                    pl.ds(other_x * m + my_y * half + i * ck, ck), :
                ],
                dst_ref=out_ref.at[
                    pl.ds(other_x * m + my_y * half + i * ck, ck), :
                ],
                send_sem=p2_send.at[i],
                recv_sem=p2_recv.at[i],
                device_id=(my_x, other_y),
                device_id_type=pl.DeviceIdType.MESH,
            )
            rdma.start()
            p2.append(rdma)

        for i in range(N_CHUNKS):
            p2[i].wait_recv()
        for i in range(N_CHUNKS):
            p1[i].wait_send()
            p2[i].wait_send()

    return pl.pallas_call(
        body,
        out_shape=jax.ShapeDtypeStruct((m_out, n_out), x.dtype),
        in_specs=[pl.BlockSpec(memory_space=pltpu.VMEM)],
        out_specs=pl.BlockSpec(memory_space=pltpu.VMEM),
        scratch_shapes=[
            pltpu.SemaphoreType.DMA((N_CHUNKS,)),
            pltpu.SemaphoreType.DMA((N_CHUNKS,)),
            pltpu.SemaphoreType.DMA((N_CHUNKS,)),
            pltpu.SemaphoreType.DMA((N_CHUNKS,)),
        ],
        compiler_params=pltpu.CompilerParams(collective_id=0),
    )(x)


# device time: 37231 ns/iter; 1.4840x vs baseline; 1.0231x over previous
import jax
import jax.numpy as jnp
from jax import lax
from jax.experimental import pallas as pl
from jax.experimental.pallas import tpu as pltpu

X_SIZE = 2
N_CHUNKS = 16


def kernel(x):
    m, n = x.shape
    n_out = n // X_SIZE
    m_out = m * X_SIZE
    half = m // 2
    ck = half // N_CHUNKS

    def body(x_ref, out_ref, local_sem, p1_send, p1_recv, p2_send, p2_recv):
        my_x = lax.axis_index("x")
        my_y = lax.axis_index("y")
        other_x = 1 - my_x
        other_y = 1 - my_y

        barrier_sem = pltpu.get_barrier_semaphore()
        for dev in [(other_x, my_y), (my_x, other_y)]:
            pl.semaphore_signal(
                barrier_sem, inc=1,
                device_id=dev, device_id_type=pl.DeviceIdType.MESH,
            )
        pl.semaphore_wait(barrier_sem, 2)

        p1 = []
        for i in range(N_CHUNKS):
            rdma = pltpu.make_async_remote_copy(
                src_ref=x_ref.at[
                    pl.ds(my_y * half + i * ck, ck),
                    pl.ds(other_x * n_out, n_out),
                ],
                dst_ref=out_ref.at[pl.ds(my_x * m + my_y * half + i * ck, ck), :],
                send_sem=p1_send.at[i],
                recv_sem=p1_recv.at[i],
                device_id=(other_x, my_y),
                device_id_type=pl.DeviceIdType.MESH,
            )
            rdma.start()
            p1.append(rdma)

        local = pltpu.make_async_copy(
            x_ref.at[:, pl.ds(my_x * n_out, n_out)],
            out_ref.at[pl.ds(my_x * m, m), :],
            local_sem,
        )
        local.start()

        p2 = []
        for i in range(N_CHUNKS):
            p1[i].wait_recv()
            rdma = pltpu.make_async_remote_copy(
                src_ref=out_ref.at[
                    pl.ds(other_x * m + my_y * half + i * ck, ck), :
                ],
                dst_ref=out_ref.at[
                    pl.ds(other_x * m + my_y * half + i * ck, ck), :
                ],
                send_sem=p2_send.at[i],
                recv_sem=p2_recv.at[i],
                device_id=(my_x, other_y),
                device_id_type=pl.DeviceIdType.MESH,
            )
            rdma.start()
            p2.append(rdma)

        local.wait()
        for i in range(N_CHUNKS):
            p2[i].wait_recv()
        for i in range(N_CHUNKS):
            p1[i].wait_send()
            p2[i].wait_send()

    return pl.pallas_call(
        body,
        out_shape=jax.ShapeDtypeStruct((m_out, n_out), x.dtype),
        in_specs=[pl.BlockSpec(memory_space=pl.ANY)],
        out_specs=pl.BlockSpec(memory_space=pl.ANY),
        scratch_shapes=[
            pltpu.SemaphoreType.DMA,
            pltpu.SemaphoreType.DMA((N_CHUNKS,)),
            pltpu.SemaphoreType.DMA((N_CHUNKS,)),
            pltpu.SemaphoreType.DMA((N_CHUNKS,)),
            pltpu.SemaphoreType.DMA((N_CHUNKS,)),
        ],
        compiler_params=pltpu.CompilerParams(collective_id=0),
    )(x)
